# baseline (speedup 1.0000x reference)
import functools

import jax
import jax.numpy as jnp
from jax import lax
from jax.experimental import pallas as pl
from jax.experimental.pallas import tpu as pltpu
from jax.experimental.pallas import tpu_sc as plsc

N = 10000
F = 128
H = 128
HW = 64
E = 320000

NUM_TILES = 16
CHUNK = 128
NBUF = 2
NIDX = 40
CPT = 160
NGRP = CPT // NIDX
EPAD = CPT * NUM_TILES * CHUNK
PADROW = N
NACC = 10240
STRIPE = NACC // NUM_TILES
LANES = 16


def _matmul_split(x, w, rows_blk):
    def body(x_ref, w_ref, lo_ref, hi_ref):
        res = jnp.dot(x_ref[...], w_ref[...], preferred_element_type=jnp.float32)
        lo_ref[...] = res[:, :HW]
        hi_ref[...] = res[:, HW:]
    grid = (N // rows_blk,)
    return pl.pallas_call(
        body,
        grid=grid,
        in_specs=[
            pl.BlockSpec((rows_blk, F), lambda r: (r, 0)),
            pl.BlockSpec((F, H), lambda r: (0, 0)),
        ],
        out_specs=[
            pl.BlockSpec((rows_blk, HW), lambda r: (r, 0)),
            pl.BlockSpec((rows_blk, HW), lambda r: (r, 0)),
        ],
        out_shape=[
            jax.ShapeDtypeStruct((N, HW), jnp.float32),
            jax.ShapeDtypeStruct((N, HW), jnp.float32),
        ],
    )(x, w)


def _bias_relu_matmul_split(a_lo, a_hi, b, w, rows_blk):
    def body(lo_ref, hi_ref, b_ref, w_ref, olo_ref, ohi_ref):
        a = jnp.concatenate([lo_ref[...], hi_ref[...]], axis=1)
        h = jnp.maximum(a + b_ref[...], 0.0)
        res = jnp.dot(h, w_ref[...], preferred_element_type=jnp.float32)
        olo_ref[...] = res[:, :HW]
        ohi_ref[...] = res[:, HW:]
    grid = (N // rows_blk,)
    return pl.pallas_call(
        body,
        grid=grid,
        in_specs=[
            pl.BlockSpec((rows_blk, HW), lambda r: (r, 0)),
            pl.BlockSpec((rows_blk, HW), lambda r: (r, 0)),
            pl.BlockSpec((1, H), lambda r: (0, 0)),
            pl.BlockSpec((H, H), lambda r: (0, 0)),
        ],
        out_specs=[
            pl.BlockSpec((rows_blk, HW), lambda r: (r, 0)),
            pl.BlockSpec((rows_blk, HW), lambda r: (r, 0)),
        ],
        out_shape=[
            jax.ShapeDtypeStruct((N, HW), jnp.float32),
            jax.ShapeDtypeStruct((N, HW), jnp.float32),
        ],
    )(a_lo, a_hi, b.reshape(1, H), w)


def _bias_add_cat(a_lo, a_hi, b, rows_blk):
    def body(lo_ref, hi_ref, b_ref, o_ref):
        o_ref[...] = jnp.concatenate([lo_ref[...], hi_ref[...]], axis=1) + b_ref[...]
    grid = (N // rows_blk,)
    return pl.pallas_call(
        body,
        grid=grid,
        in_specs=[
            pl.BlockSpec((rows_blk, HW), lambda r: (r, 0)),
            pl.BlockSpec((rows_blk, HW), lambda r: (r, 0)),
            pl.BlockSpec((1, H), lambda r: (0, 0)),
        ],
        out_specs=pl.BlockSpec((rows_blk, H), lambda r: (r, 0)),
        out_shape=jax.ShapeDtypeStruct((N, H), jnp.float32),
    )(a_lo, a_hi, b.reshape(1, H))


def _sc_conv(mlo_d, mhi_d, mlo_s, mhi_s, eid, eis):
    mesh = plsc.VectorSubcoreMesh(core_axis_name="c", subcore_axis_name="s")

    @functools.partial(
        pl.kernel,
        out_type=tuple(jax.ShapeDtypeStruct((N, HW), jnp.float32) for _ in range(4)),
        mesh=mesh,
        compiler_params=pltpu.CompilerParams(use_tc_tiling_on_sc=False),
        scratch_types=[
            pltpu.VMEM_SHARED((NACC, HW), jnp.float32),
            pltpu.VMEM_SHARED((NACC, HW), jnp.float32),
            [pltpu.VMEM((CHUNK, HW), jnp.float32)] * NBUF,
            pltpu.VMEM((NIDX, CHUNK), jnp.int32),
            pltpu.VMEM((NIDX, CHUNK), jnp.int32),
            [pltpu.SemaphoreType.DMA] * NBUF,
        ],
    )
    def conv_kernel(mlo_d_hbm, mhi_d_hbm, mlo_s_hbm, mhi_s_hbm, eid_hbm, eis_hbm,
                    alo_d_hbm, ahi_d_hbm, alo_s_hbm, ahi_s_hbm,
                    msp, acc, rows, src_idx, dst_idx, gsems):
        c = lax.axis_index("c")
        s = lax.axis_index("s")

        def zrow(i, _):
            def zlane(j, _):
                rows[0][i, pl.ds(j * LANES, LANES)] = jnp.zeros((LANES,), jnp.float32)
                return 0
            return lax.fori_loop(0, HW // LANES, zlane, 0)
        lax.fori_loop(0, CHUNK, zrow, 0)

        def one_pass(m_hbm, ei_hbm, out_hbm):
            sbase = s * STRIPE

            @pl.when(s < 15)
            def _():
                pltpu.sync_copy(m_hbm.at[pl.ds(sbase, STRIPE)],
                                msp.at[pl.ds(sbase, STRIPE)])

            @pl.when(s == 15)
            def _():
                pltpu.sync_copy(m_hbm.at[pl.ds(9600, 400)],
                                msp.at[pl.ds(9600, 400)])

            def zcopy(k, _):
                pltpu.sync_copy(rows[0], acc.at[pl.ds(sbase + k * CHUNK, CHUNK)])
                return 0
            lax.fori_loop(0, STRIPE // CHUNK, zcopy, 0)
            plsc.subcore_barrier()

            def gather(k, b):
                return pltpu.make_async_copy(msp.at[src_idx.at[k]],
                                             rows[b], gsems[b])

            def grp(gi, _):
                pltpu.sync_copy(ei_hbm.at[0, s, pl.ds(gi * NIDX, NIDX)], src_idx)
                pltpu.sync_copy(ei_hbm.at[1, s, pl.ds(gi * NIDX, NIDX)], dst_idx)
                for b in range(NBUF):
                    gather(b, b).start()

                def inner(t, _):
                    for b in range(NBUF):
                        k = t * NBUF + b
                        gather(k, b).wait()
                        pltpu.sync_copy(rows[b], acc.at[dst_idx.at[k]], add=True)

                        @pl.when(k + NBUF < NIDX)
                        def _():
                            gather(k + NBUF, b).start()
                    return 0
                lax.fori_loop(0, NIDX // NBUF, inner, 0)
                return 0
            lax.fori_loop(0, NGRP, grp, 0)
            plsc.subcore_barrier()

            @pl.when(s < 15)
            def _():
                pltpu.sync_copy(acc.at[pl.ds(sbase, STRIPE)],
                                out_hbm.at[pl.ds(sbase, STRIPE)])

            @pl.when(s == 15)
            def _():
                pltpu.sync_copy(acc.at[pl.ds(9600, 400)],
                                out_hbm.at[pl.ds(9600, 400)])

            def rezrow(i, _):
                def rezlane(j, _):
                    rows[0][i, pl.ds(j * LANES, LANES)] = jnp.zeros((LANES,), jnp.float32)
                    return 0
                return lax.fori_loop(0, HW // LANES, rezlane, 0)
            lax.fori_loop(0, CHUNK, rezrow, 0)

        @pl.when(c == 0)
        def _():
            one_pass(mlo_d_hbm, eid_hbm, alo_d_hbm)
            one_pass(mhi_d_hbm, eid_hbm, ahi_d_hbm)

        @pl.when(c == 1)
        def _():
            one_pass(mlo_s_hbm, eis_hbm, alo_s_hbm)
            one_pass(mhi_s_hbm, eis_hbm, ahi_s_hbm)

    return conv_kernel(mlo_d, mhi_d, mlo_s, mhi_s, eid, eis)


def _pad_edges(ei):
    pad = EPAD - E
    pad_cols = jnp.concatenate([
        jnp.zeros((1, pad), jnp.int32),
        jnp.full((1, pad), PADROW, jnp.int32),
    ], axis=0)
    padded = jnp.concatenate([ei, pad_cols], axis=1)
    return padded.reshape(2, NUM_TILES, CPT, CHUNK)


def kernel(drug_x, drug_edge_index, dis_x, dis_edge_index,
           W1d, b1d, W2d, b2d, W1s, b1s, W2s, b2s):
    eid = _pad_edges(drug_edge_index)
    eis = _pad_edges(dis_edge_index)

    rows_blk = 1000
    m1lo_d, m1hi_d = _matmul_split(drug_x, W1d, rows_blk)
    m1lo_s, m1hi_s = _matmul_split(dis_x, W1s, rows_blk)
    a1lo_d, a1hi_d, a1lo_s, a1hi_s = _sc_conv(m1lo_d, m1hi_d, m1lo_s, m1hi_s, eid, eis)
    m2lo_d, m2hi_d = _bias_relu_matmul_split(a1lo_d, a1hi_d, b1d, W2d, rows_blk)
    m2lo_s, m2hi_s = _bias_relu_matmul_split(a1lo_s, a1hi_s, b1s, W2s, rows_blk)
    a2lo_d, a2hi_d, a2lo_s, a2hi_s = _sc_conv(m2lo_d, m2hi_d, m2lo_s, m2hi_s, eid, eis)
    emb1 = _bias_add_cat(a2lo_d, a2hi_d, b2d, rows_blk)
    emb2 = _bias_add_cat(a2lo_s, a2hi_s, b2s, rows_blk)
    return (emb1, emb2)


# speedup vs baseline: 1.6728x; 1.6728x over previous
import functools

import jax
import jax.numpy as jnp
from jax import lax
from jax.experimental import pallas as pl
from jax.experimental.pallas import tpu as pltpu
from jax.experimental.pallas import tpu_sc as plsc

N = 10000
F = 128
H = 128
E = 320000

NUM_TILES = 16
CHUNK = 128
NBUF = 2
NIDX = 40
CPT = 160
NGRP = CPT // NIDX
EPAD = CPT * NUM_TILES * CHUNK
PADROW = N
NACC = 10240
STRIPE = NACC // NUM_TILES
BLANES = 32


def _matmul_bf16(x, w, rows_blk):
    def body(x_ref, w_ref, o_ref):
        res = jnp.dot(x_ref[...], w_ref[...], preferred_element_type=jnp.float32)
        o_ref[...] = res.astype(jnp.bfloat16)
    grid = (N // rows_blk,)
    return pl.pallas_call(
        body,
        grid=grid,
        in_specs=[
            pl.BlockSpec((rows_blk, F), lambda r: (r, 0)),
            pl.BlockSpec((F, H), lambda r: (0, 0)),
        ],
        out_specs=pl.BlockSpec((rows_blk, H), lambda r: (r, 0)),
        out_shape=jax.ShapeDtypeStruct((N, H), jnp.bfloat16),
    )(x, w)


def _bias_relu_matmul_bf16(a, b, w, rows_blk):
    def body(a_ref, b_ref, w_ref, o_ref):
        h = jnp.maximum(a_ref[...].astype(jnp.float32) + b_ref[...], 0.0)
        res = jnp.dot(h, w_ref[...], preferred_element_type=jnp.float32)
        o_ref[...] = res.astype(jnp.bfloat16)
    grid = (N // rows_blk,)
    return pl.pallas_call(
        body,
        grid=grid,
        in_specs=[
            pl.BlockSpec((rows_blk, H), lambda r: (r, 0)),
            pl.BlockSpec((1, H), lambda r: (0, 0)),
            pl.BlockSpec((H, H), lambda r: (0, 0)),
        ],
        out_specs=pl.BlockSpec((rows_blk, H), lambda r: (r, 0)),
        out_shape=jax.ShapeDtypeStruct((N, H), jnp.bfloat16),
    )(a, b.reshape(1, H), w)


def _bias_add_f32(a, b, rows_blk):
    def body(a_ref, b_ref, o_ref):
        o_ref[...] = a_ref[...].astype(jnp.float32) + b_ref[...]
    grid = (N // rows_blk,)
    return pl.pallas_call(
        body,
        grid=grid,
        in_specs=[
            pl.BlockSpec((rows_blk, H), lambda r: (r, 0)),
            pl.BlockSpec((1, H), lambda r: (0, 0)),
        ],
        out_specs=pl.BlockSpec((rows_blk, H), lambda r: (r, 0)),
        out_shape=jax.ShapeDtypeStruct((N, H), jnp.float32),
    )(a, b.reshape(1, H))


def _sc_conv(m_d, m_s, eid, eis):
    mesh = plsc.VectorSubcoreMesh(core_axis_name="c", subcore_axis_name="s")

    @functools.partial(
        pl.kernel,
        out_type=(
            jax.ShapeDtypeStruct((N, H), jnp.bfloat16),
            jax.ShapeDtypeStruct((N, H), jnp.bfloat16),
        ),
        mesh=mesh,
        compiler_params=pltpu.CompilerParams(use_tc_tiling_on_sc=False),
        scratch_types=[
            pltpu.VMEM_SHARED((NACC, H), jnp.bfloat16),
            pltpu.VMEM_SHARED((NACC, H), jnp.bfloat16),
            [pltpu.VMEM((CHUNK, H), jnp.bfloat16)] * NBUF,
            pltpu.VMEM((NIDX, CHUNK), jnp.int32),
            pltpu.VMEM((NIDX, CHUNK), jnp.int32),
            [pltpu.SemaphoreType.DMA] * NBUF,
        ],
    )
    def conv_kernel(m_d_hbm, m_s_hbm, eid_hbm, eis_hbm, agg_d_hbm, agg_s_hbm,
                    msp, acc, rows, src_idx, dst_idx, gsems):
        c = lax.axis_index("c")
        s = lax.axis_index("s")

        def zrow(i, _):
            def zlane(j, _):
                rows[0][i, pl.ds(j * BLANES, BLANES)] = jnp.zeros((BLANES,), jnp.bfloat16)
                return 0
            return lax.fori_loop(0, H // BLANES, zlane, 0)
        lax.fori_loop(0, CHUNK, zrow, 0)

        def run(m_hbm, ei_hbm, out_hbm):
            sbase = s * STRIPE

            @pl.when(s < 15)
            def _():
                pltpu.sync_copy(m_hbm.at[pl.ds(sbase, STRIPE)],
                                msp.at[pl.ds(sbase, STRIPE)])

            @pl.when(s == 15)
            def _():
                pltpu.sync_copy(m_hbm.at[pl.ds(9600, 400)],
                                msp.at[pl.ds(9600, 400)])

            def zcopy(k, _):
                pltpu.sync_copy(rows[0], acc.at[pl.ds(sbase + k * CHUNK, CHUNK)])
                return 0
            lax.fori_loop(0, STRIPE // CHUNK, zcopy, 0)
            plsc.subcore_barrier()

            def gather(k, b):
                return pltpu.make_async_copy(msp.at[src_idx.at[k]],
                                             rows[b], gsems[b])

            def grp(gi, _):
                pltpu.sync_copy(ei_hbm.at[0, s, pl.ds(gi * NIDX, NIDX)], src_idx)
                pltpu.sync_copy(ei_hbm.at[1, s, pl.ds(gi * NIDX, NIDX)], dst_idx)
                for b in range(NBUF):
                    gather(b, b).start()

                def inner(t, _):
                    for b in range(NBUF):
                        k = t * NBUF + b
                        gather(k, b).wait()
                        pltpu.sync_copy(rows[b], acc.at[dst_idx.at[k]], add=True)

                        @pl.when(k + NBUF < NIDX)
                        def _():
                            gather(k + NBUF, b).start()
                    return 0
                lax.fori_loop(0, NIDX // NBUF, inner, 0)
                return 0
            lax.fori_loop(0, NGRP, grp, 0)
            plsc.subcore_barrier()

            @pl.when(s < 15)
            def _():
                pltpu.sync_copy(acc.at[pl.ds(sbase, STRIPE)],
                                out_hbm.at[pl.ds(sbase, STRIPE)])

            @pl.when(s == 15)
            def _():
                pltpu.sync_copy(acc.at[pl.ds(9600, 400)],
                                out_hbm.at[pl.ds(9600, 400)])

        @pl.when(c == 0)
        def _():
            run(m_d_hbm, eid_hbm, agg_d_hbm)

        @pl.when(c == 1)
        def _():
            run(m_s_hbm, eis_hbm, agg_s_hbm)

    return conv_kernel(m_d, m_s, eid, eis)


def _pad_edges(ei):
    pad = EPAD - E
    pad_cols = jnp.concatenate([
        jnp.zeros((1, pad), jnp.int32),
        jnp.full((1, pad), PADROW, jnp.int32),
    ], axis=0)
    padded = jnp.concatenate([ei, pad_cols], axis=1)
    return padded.reshape(2, NUM_TILES, CPT, CHUNK)


def kernel(drug_x, drug_edge_index, dis_x, dis_edge_index,
           W1d, b1d, W2d, b2d, W1s, b1s, W2s, b2s):
    eid = _pad_edges(drug_edge_index)
    eis = _pad_edges(dis_edge_index)

    rows_blk = 1000
    m1_d = _matmul_bf16(drug_x, W1d, rows_blk)
    m1_s = _matmul_bf16(dis_x, W1s, rows_blk)
    a1_d, a1_s = _sc_conv(m1_d, m1_s, eid, eis)
    m2_d = _bias_relu_matmul_bf16(a1_d, b1d, W2d, rows_blk)
    m2_s = _bias_relu_matmul_bf16(a1_s, b1s, W2s, rows_blk)
    a2_d, a2_s = _sc_conv(m2_d, m2_s, eid, eis)
    emb1 = _bias_add_f32(a2_d, b2d, rows_blk)
    emb2 = _bias_add_f32(a2_s, b2s, rows_blk)
    return (emb1, emb2)


# speedup vs baseline: 1.7629x; 1.0538x over previous
import functools

import jax
import jax.numpy as jnp
from jax import lax
from jax.experimental import pallas as pl
from jax.experimental.pallas import tpu as pltpu
from jax.experimental.pallas import tpu_sc as plsc

N = 10000
F = 128
H = 128
E = 320000

NUM_TILES = 16
CHUNK = 128
NBUF = 2
NIDX = 40
CPT = 160
NGRP = CPT // NIDX
EPAD = CPT * NUM_TILES * CHUNK
PADROW = N
NACC = 10240
STRIPE = NACC // NUM_TILES
BLANES = 32


def _matmul2_bf16(xd, xs, wd, ws, rows_blk):
    def body(xd_ref, xs_ref, wd_ref, ws_ref, od_ref, os_ref):
        od_ref[...] = jnp.dot(xd_ref[...], wd_ref[...],
                              preferred_element_type=jnp.float32).astype(jnp.bfloat16)
        os_ref[...] = jnp.dot(xs_ref[...], ws_ref[...],
                              preferred_element_type=jnp.float32).astype(jnp.bfloat16)
    grid = (N // rows_blk,)
    x_spec = pl.BlockSpec((rows_blk, F), lambda r: (r, 0))
    w_spec = pl.BlockSpec((F, H), lambda r: (0, 0))
    o_spec = pl.BlockSpec((rows_blk, H), lambda r: (r, 0))
    o_type = jax.ShapeDtypeStruct((N, H), jnp.bfloat16)
    return pl.pallas_call(
        body, grid=grid,
        in_specs=[x_spec, x_spec, w_spec, w_spec],
        out_specs=[o_spec, o_spec],
        out_shape=[o_type, o_type],
    )(xd, xs, wd, ws)


def _bias_relu_matmul2_bf16(ad, as_, bd, bs, wd, ws, rows_blk):
    def body(ad_ref, as_ref, bd_ref, bs_ref, wd_ref, ws_ref, od_ref, os_ref):
        hd = jnp.maximum(ad_ref[...].astype(jnp.float32) + bd_ref[...], 0.0)
        od_ref[...] = jnp.dot(hd, wd_ref[...],
                              preferred_element_type=jnp.float32).astype(jnp.bfloat16)
        hs = jnp.maximum(as_ref[...].astype(jnp.float32) + bs_ref[...], 0.0)
        os_ref[...] = jnp.dot(hs, ws_ref[...],
                              preferred_element_type=jnp.float32).astype(jnp.bfloat16)
    grid = (N // rows_blk,)
    a_spec = pl.BlockSpec((rows_blk, H), lambda r: (r, 0))
    b_spec = pl.BlockSpec((1, H), lambda r: (0, 0))
    w_spec = pl.BlockSpec((H, H), lambda r: (0, 0))
    o_spec = pl.BlockSpec((rows_blk, H), lambda r: (r, 0))
    o_type = jax.ShapeDtypeStruct((N, H), jnp.bfloat16)
    return pl.pallas_call(
        body, grid=grid,
        in_specs=[a_spec, a_spec, b_spec, b_spec, w_spec, w_spec],
        out_specs=[o_spec, o_spec],
        out_shape=[o_type, o_type],
    )(ad, as_, bd.reshape(1, H), bs.reshape(1, H), wd, ws)


def _bias_add2_f32(ad, as_, bd, bs, rows_blk):
    def body(ad_ref, as_ref, bd_ref, bs_ref, od_ref, os_ref):
        od_ref[...] = ad_ref[...].astype(jnp.float32) + bd_ref[...]
        os_ref[...] = as_ref[...].astype(jnp.float32) + bs_ref[...]
    grid = (N // rows_blk,)
    a_spec = pl.BlockSpec((rows_blk, H), lambda r: (r, 0))
    b_spec = pl.BlockSpec((1, H), lambda r: (0, 0))
    o_spec = pl.BlockSpec((rows_blk, H), lambda r: (r, 0))
    o_type = jax.ShapeDtypeStruct((N, H), jnp.float32)
    return pl.pallas_call(
        body, grid=grid,
        in_specs=[a_spec, a_spec, b_spec, b_spec],
        out_specs=[o_spec, o_spec],
        out_shape=[o_type, o_type],
    )(ad, as_, bd.reshape(1, H), bs.reshape(1, H))


def _sc_conv(m_d, m_s, eid, eis):
    mesh = plsc.VectorSubcoreMesh(core_axis_name="c", subcore_axis_name="s")

    @functools.partial(
        pl.kernel,
        out_type=(
            jax.ShapeDtypeStruct((N, H), jnp.bfloat16),
            jax.ShapeDtypeStruct((N, H), jnp.bfloat16),
        ),
        mesh=mesh,
        compiler_params=pltpu.CompilerParams(use_tc_tiling_on_sc=False),
        scratch_types=[
            pltpu.VMEM_SHARED((NACC, H), jnp.bfloat16),
            pltpu.VMEM_SHARED((NACC, H), jnp.bfloat16),
            [pltpu.VMEM((CHUNK, H), jnp.bfloat16)] * NBUF,
            pltpu.VMEM((NIDX, CHUNK), jnp.int32),
            pltpu.VMEM((NIDX, CHUNK), jnp.int32),
            [pltpu.SemaphoreType.DMA] * NBUF,
        ],
    )
    def conv_kernel(m_d_hbm, m_s_hbm, eid_hbm, eis_hbm, agg_d_hbm, agg_s_hbm,
                    msp, acc, rows, src_idx, dst_idx, gsems):
        c = lax.axis_index("c")
        s = lax.axis_index("s")

        def zrow(i, _):
            def zlane(j, _):
                rows[0][i, pl.ds(j * BLANES, BLANES)] = jnp.zeros((BLANES,), jnp.bfloat16)
                return 0
            return lax.fori_loop(0, H // BLANES, zlane, 0)
        lax.fori_loop(0, CHUNK, zrow, 0)

        def run(m_hbm, ei_hbm, out_hbm):
            sbase = s * STRIPE

            @pl.when(s < 15)
            def _():
                pltpu.sync_copy(m_hbm.at[pl.ds(sbase, STRIPE)],
                                msp.at[pl.ds(sbase, STRIPE)])

            @pl.when(s == 15)
            def _():
                pltpu.sync_copy(m_hbm.at[pl.ds(9600, 400)],
                                msp.at[pl.ds(9600, 400)])

            def zcopy(k, _):
                pltpu.sync_copy(rows[0], acc.at[pl.ds(sbase + k * CHUNK, CHUNK)])
                return 0
            lax.fori_loop(0, STRIPE // CHUNK, zcopy, 0)
            plsc.subcore_barrier()

            def gather(k, b):
                return pltpu.make_async_copy(msp.at[src_idx.at[k]],
                                             rows[b], gsems[b])

            def grp(gi, _):
                pltpu.sync_copy(ei_hbm.at[0, s, pl.ds(gi * NIDX, NIDX)], src_idx)
                pltpu.sync_copy(ei_hbm.at[1, s, pl.ds(gi * NIDX, NIDX)], dst_idx)
                for b in range(NBUF):
                    gather(b, b).start()

                def inner(t, _):
                    for b in range(NBUF):
                        k = t * NBUF + b
                        gather(k, b).wait()
                        pltpu.sync_copy(rows[b], acc.at[dst_idx.at[k]], add=True)

                        @pl.when(k + NBUF < NIDX)
                        def _():
                            gather(k + NBUF, b).start()
                    return 0
                lax.fori_loop(0, NIDX // NBUF, inner, 0)
                return 0
            lax.fori_loop(0, NGRP, grp, 0)
            plsc.subcore_barrier()

            @pl.when(s < 15)
            def _():
                pltpu.sync_copy(acc.at[pl.ds(sbase, STRIPE)],
                                out_hbm.at[pl.ds(sbase, STRIPE)])

            @pl.when(s == 15)
            def _():
                pltpu.sync_copy(acc.at[pl.ds(9600, 400)],
                                out_hbm.at[pl.ds(9600, 400)])

        @pl.when(c == 0)
        def _():
            run(m_d_hbm, eid_hbm, agg_d_hbm)

        @pl.when(c == 1)
        def _():
            run(m_s_hbm, eis_hbm, agg_s_hbm)

    return conv_kernel(m_d, m_s, eid, eis)


def _pad_edges(ei):
    pad = EPAD - E
    pad_cols = jnp.concatenate([
        jnp.zeros((1, pad), jnp.int32),
        jnp.full((1, pad), PADROW, jnp.int32),
    ], axis=0)
    padded = jnp.concatenate([ei, pad_cols], axis=1)
    return padded.reshape(2, NUM_TILES, CPT, CHUNK)


def kernel(drug_x, drug_edge_index, dis_x, dis_edge_index,
           W1d, b1d, W2d, b2d, W1s, b1s, W2s, b2s):
    eid = _pad_edges(drug_edge_index)
    eis = _pad_edges(dis_edge_index)

    rows_blk = 1000
    m1_d, m1_s = _matmul2_bf16(drug_x, dis_x, W1d, W1s, rows_blk)
    a1_d, a1_s = _sc_conv(m1_d, m1_s, eid, eis)
    m2_d, m2_s = _bias_relu_matmul2_bf16(a1_d, a1_s, b1d, b1s, W2d, W2s, rows_blk)
    a2_d, a2_s = _sc_conv(m2_d, m2_s, eid, eis)
    emb1, emb2 = _bias_add2_f32(a2_d, a2_s, b2d, b2s, rows_blk)
    return (emb1, emb2)


# speedup vs baseline: 1.8140x; 1.0290x over previous
import functools

import jax
import jax.numpy as jnp
from jax import lax
from jax.experimental import pallas as pl
from jax.experimental.pallas import tpu as pltpu
from jax.experimental.pallas import tpu_sc as plsc

N = 10000
F = 128
H = 128
E = 320000

NUM_TILES = 16
CHUNK = 128
NBUF = 2
NIDX = 80
CPT = 160
NGRP = CPT // NIDX
EPAD = CPT * NUM_TILES * CHUNK
PADROW = N
NACC = 10240
STRIPE = NACC // NUM_TILES
BLANES = 32


def _matmul2_bf16(xd, xs, wd, ws, rows_blk):
    def body(xd_ref, xs_ref, wd_ref, ws_ref, od_ref, os_ref):
        od_ref[...] = jnp.dot(xd_ref[...], wd_ref[...],
                              preferred_element_type=jnp.float32).astype(jnp.bfloat16)
        os_ref[...] = jnp.dot(xs_ref[...], ws_ref[...],
                              preferred_element_type=jnp.float32).astype(jnp.bfloat16)
    grid = (N // rows_blk,)
    x_spec = pl.BlockSpec((rows_blk, F), lambda r: (r, 0))
    w_spec = pl.BlockSpec((F, H), lambda r: (0, 0))
    o_spec = pl.BlockSpec((rows_blk, H), lambda r: (r, 0))
    o_type = jax.ShapeDtypeStruct((N, H), jnp.bfloat16)
    return pl.pallas_call(
        body, grid=grid,
        in_specs=[x_spec, x_spec, w_spec, w_spec],
        out_specs=[o_spec, o_spec],
        out_shape=[o_type, o_type],
    )(xd, xs, wd, ws)


def _bias_relu_matmul2_bf16(ad, as_, bd, bs, wd, ws, rows_blk):
    def body(ad_ref, as_ref, bd_ref, bs_ref, wd_ref, ws_ref, od_ref, os_ref):
        hd = jnp.maximum(ad_ref[...].astype(jnp.float32) + bd_ref[...], 0.0)
        od_ref[...] = jnp.dot(hd, wd_ref[...],
                              preferred_element_type=jnp.float32).astype(jnp.bfloat16)
        hs = jnp.maximum(as_ref[...].astype(jnp.float32) + bs_ref[...], 0.0)
        os_ref[...] = jnp.dot(hs, ws_ref[...],
                              preferred_element_type=jnp.float32).astype(jnp.bfloat16)
    grid = (N // rows_blk,)
    a_spec = pl.BlockSpec((rows_blk, H), lambda r: (r, 0))
    b_spec = pl.BlockSpec((1, H), lambda r: (0, 0))
    w_spec = pl.BlockSpec((H, H), lambda r: (0, 0))
    o_spec = pl.BlockSpec((rows_blk, H), lambda r: (r, 0))
    o_type = jax.ShapeDtypeStruct((N, H), jnp.bfloat16)
    return pl.pallas_call(
        body, grid=grid,
        in_specs=[a_spec, a_spec, b_spec, b_spec, w_spec, w_spec],
        out_specs=[o_spec, o_spec],
        out_shape=[o_type, o_type],
    )(ad, as_, bd.reshape(1, H), bs.reshape(1, H), wd, ws)


def _bias_add2_f32(ad, as_, bd, bs, rows_blk):
    def body(ad_ref, as_ref, bd_ref, bs_ref, od_ref, os_ref):
        od_ref[...] = ad_ref[...].astype(jnp.float32) + bd_ref[...]
        os_ref[...] = as_ref[...].astype(jnp.float32) + bs_ref[...]
    grid = (N // rows_blk,)
    a_spec = pl.BlockSpec((rows_blk, H), lambda r: (r, 0))
    b_spec = pl.BlockSpec((1, H), lambda r: (0, 0))
    o_spec = pl.BlockSpec((rows_blk, H), lambda r: (r, 0))
    o_type = jax.ShapeDtypeStruct((N, H), jnp.float32)
    return pl.pallas_call(
        body, grid=grid,
        in_specs=[a_spec, a_spec, b_spec, b_spec],
        out_specs=[o_spec, o_spec],
        out_shape=[o_type, o_type],
    )(ad, as_, bd.reshape(1, H), bs.reshape(1, H))


def _sc_conv(m_d, m_s, eid, eis):
    mesh = plsc.VectorSubcoreMesh(core_axis_name="c", subcore_axis_name="s")

    @functools.partial(
        pl.kernel,
        out_type=(
            jax.ShapeDtypeStruct((N, H), jnp.bfloat16),
            jax.ShapeDtypeStruct((N, H), jnp.bfloat16),
        ),
        mesh=mesh,
        compiler_params=pltpu.CompilerParams(use_tc_tiling_on_sc=False),
        scratch_types=[
            pltpu.VMEM_SHARED((NACC, H), jnp.bfloat16),
            pltpu.VMEM_SHARED((NACC, H), jnp.bfloat16),
            [pltpu.VMEM((CHUNK, H), jnp.bfloat16)] * NBUF,
            pltpu.VMEM((NIDX, CHUNK), jnp.int32),
            pltpu.VMEM((NIDX, CHUNK), jnp.int32),
            [pltpu.SemaphoreType.DMA] * NBUF,
        ],
    )
    def conv_kernel(m_d_hbm, m_s_hbm, eid_hbm, eis_hbm, agg_d_hbm, agg_s_hbm,
                    msp, acc, rows, src_idx, dst_idx, gsems):
        c = lax.axis_index("c")
        s = lax.axis_index("s")

        def zrow(i, _):
            def zlane(j, _):
                rows[0][i, pl.ds(j * BLANES, BLANES)] = jnp.zeros((BLANES,), jnp.bfloat16)
                return 0
            return lax.fori_loop(0, H // BLANES, zlane, 0)
        lax.fori_loop(0, CHUNK, zrow, 0)

        def run(m_hbm, ei_hbm, out_hbm):
            sbase = s * STRIPE

            @pl.when(s < 15)
            def _():
                pltpu.sync_copy(m_hbm.at[pl.ds(sbase, STRIPE)],
                                msp.at[pl.ds(sbase, STRIPE)])

            @pl.when(s == 15)
            def _():
                pltpu.sync_copy(m_hbm.at[pl.ds(9600, 400)],
                                msp.at[pl.ds(9600, 400)])

            def zcopy(k, _):
                pltpu.sync_copy(rows[0], acc.at[pl.ds(sbase + k * CHUNK, CHUNK)])
                return 0
            lax.fori_loop(0, STRIPE // CHUNK, zcopy, 0)
            plsc.subcore_barrier()

            def gather(k, b):
                return pltpu.make_async_copy(msp.at[src_idx.at[k]],
                                             rows[b], gsems[b])

            def grp(gi, _):
                pltpu.sync_copy(ei_hbm.at[0, s, pl.ds(gi * NIDX, NIDX)], src_idx)
                pltpu.sync_copy(ei_hbm.at[1, s, pl.ds(gi * NIDX, NIDX)], dst_idx)
                for b in range(NBUF):
                    gather(b, b).start()

                def inner(t, _):
                    for b in range(NBUF):
                        k = t * NBUF + b
                        gather(k, b).wait()
                        pltpu.sync_copy(rows[b], acc.at[dst_idx.at[k]], add=True)

                        @pl.when(k + NBUF < NIDX)
                        def _():
                            gather(k + NBUF, b).start()
                    return 0
                lax.fori_loop(0, NIDX // NBUF, inner, 0)
                return 0
            lax.fori_loop(0, NGRP, grp, 0)
            plsc.subcore_barrier()

            @pl.when(s < 15)
            def _():
                pltpu.sync_copy(acc.at[pl.ds(sbase, STRIPE)],
                                out_hbm.at[pl.ds(sbase, STRIPE)])

            @pl.when(s == 15)
            def _():
                pltpu.sync_copy(acc.at[pl.ds(9600, 400)],
                                out_hbm.at[pl.ds(9600, 400)])

        @pl.when(c == 0)
        def _():
            run(m_d_hbm, eid_hbm, agg_d_hbm)

        @pl.when(c == 1)
        def _():
            run(m_s_hbm, eis_hbm, agg_s_hbm)

    return conv_kernel(m_d, m_s, eid, eis)


def _pad_edges(ei):
    pad = EPAD - E
    pad_cols = jnp.concatenate([
        jnp.zeros((1, pad), jnp.int32),
        jnp.full((1, pad), PADROW, jnp.int32),
    ], axis=0)
    padded = jnp.concatenate([ei, pad_cols], axis=1)
    return padded.reshape(2, NUM_TILES, CPT, CHUNK)


def kernel(drug_x, drug_edge_index, dis_x, dis_edge_index,
           W1d, b1d, W2d, b2d, W1s, b1s, W2s, b2s):
    eid = _pad_edges(drug_edge_index)
    eis = _pad_edges(dis_edge_index)

    rows_blk = 1000
    m1_d, m1_s = _matmul2_bf16(drug_x, dis_x, W1d, W1s, rows_blk)
    a1_d, a1_s = _sc_conv(m1_d, m1_s, eid, eis)
    m2_d, m2_s = _bias_relu_matmul2_bf16(a1_d, a1_s, b1d, b1s, W2d, W2s, rows_blk)
    a2_d, a2_s = _sc_conv(m2_d, m2_s, eid, eis)
    emb1, emb2 = _bias_add2_f32(a2_d, a2_s, b2d, b2s, rows_blk)
    return (emb1, emb2)


# speedup vs baseline: 1.8440x; 1.0166x over previous
import functools

import jax
import jax.numpy as jnp
from jax import lax
from jax.experimental import pallas as pl
from jax.experimental.pallas import tpu as pltpu
from jax.experimental.pallas import tpu_sc as plsc

N = 10000
F = 128
H = 128
E = 320000

NUM_TILES = 16
CHUNK = 128
NBUF = 2
NIDX = 80
CPT = 160
NGRP = CPT // NIDX
EPAD = CPT * NUM_TILES * CHUNK
PADROW = N
NACC = 10240
STRIPE = NACC // NUM_TILES
BLANES = 32


def _matmul2_bf16(xd, xs, wd, ws, rows_blk):
    def body(xd_ref, xs_ref, wd_ref, ws_ref, od_ref, os_ref):
        od_ref[...] = jnp.dot(xd_ref[...], wd_ref[...],
                              preferred_element_type=jnp.float32).astype(jnp.bfloat16)
        os_ref[...] = jnp.dot(xs_ref[...], ws_ref[...],
                              preferred_element_type=jnp.float32).astype(jnp.bfloat16)
    grid = (N // rows_blk,)
    x_spec = pl.BlockSpec((rows_blk, F), lambda r: (r, 0))
    w_spec = pl.BlockSpec((F, H), lambda r: (0, 0))
    o_spec = pl.BlockSpec((rows_blk, H), lambda r: (r, 0))
    o_type = jax.ShapeDtypeStruct((N, H), jnp.bfloat16)
    return pl.pallas_call(
        body, grid=grid,
        in_specs=[x_spec, x_spec, w_spec, w_spec],
        out_specs=[o_spec, o_spec],
        out_shape=[o_type, o_type],
    )(xd, xs, wd, ws)


def _bias_relu_matmul2_bf16(ad, as_, bd, bs, wd, ws, rows_blk):
    def body(ad_ref, as_ref, bd_ref, bs_ref, wd_ref, ws_ref, od_ref, os_ref):
        hd = jnp.maximum(ad_ref[...].astype(jnp.float32) + bd_ref[...], 0.0)
        od_ref[...] = jnp.dot(hd, wd_ref[...],
                              preferred_element_type=jnp.float32).astype(jnp.bfloat16)
        hs = jnp.maximum(as_ref[...].astype(jnp.float32) + bs_ref[...], 0.0)
        os_ref[...] = jnp.dot(hs, ws_ref[...],
                              preferred_element_type=jnp.float32).astype(jnp.bfloat16)
    grid = (N // rows_blk,)
    a_spec = pl.BlockSpec((rows_blk, H), lambda r: (r, 0))
    b_spec = pl.BlockSpec((1, H), lambda r: (0, 0))
    w_spec = pl.BlockSpec((H, H), lambda r: (0, 0))
    o_spec = pl.BlockSpec((rows_blk, H), lambda r: (r, 0))
    o_type = jax.ShapeDtypeStruct((N, H), jnp.bfloat16)
    return pl.pallas_call(
        body, grid=grid,
        in_specs=[a_spec, a_spec, b_spec, b_spec, w_spec, w_spec],
        out_specs=[o_spec, o_spec],
        out_shape=[o_type, o_type],
    )(ad, as_, bd.reshape(1, H), bs.reshape(1, H), wd, ws)


def _bias_add2_f32(ad, as_, bd, bs, rows_blk):
    def body(ad_ref, as_ref, bd_ref, bs_ref, od_ref, os_ref):
        od_ref[...] = ad_ref[...].astype(jnp.float32) + bd_ref[...]
        os_ref[...] = as_ref[...].astype(jnp.float32) + bs_ref[...]
    grid = (N // rows_blk,)
    a_spec = pl.BlockSpec((rows_blk, H), lambda r: (r, 0))
    b_spec = pl.BlockSpec((1, H), lambda r: (0, 0))
    o_spec = pl.BlockSpec((rows_blk, H), lambda r: (r, 0))
    o_type = jax.ShapeDtypeStruct((N, H), jnp.float32)
    return pl.pallas_call(
        body, grid=grid,
        in_specs=[a_spec, a_spec, b_spec, b_spec],
        out_specs=[o_spec, o_spec],
        out_shape=[o_type, o_type],
    )(ad, as_, bd.reshape(1, H), bs.reshape(1, H))


def _sc_conv(m_d, m_s, eid, eis):
    mesh = plsc.VectorSubcoreMesh(core_axis_name="c", subcore_axis_name="s")

    @functools.partial(
        pl.kernel,
        out_type=(
            jax.ShapeDtypeStruct((N, H), jnp.bfloat16),
            jax.ShapeDtypeStruct((N, H), jnp.bfloat16),
        ),
        mesh=mesh,
        compiler_params=pltpu.CompilerParams(use_tc_tiling_on_sc=False),
        scratch_types=[
            pltpu.VMEM_SHARED((NACC, H), jnp.bfloat16),
            pltpu.VMEM_SHARED((NACC, H), jnp.bfloat16),
            [pltpu.VMEM((CHUNK, H), jnp.bfloat16)] * NBUF,
            pltpu.VMEM((NIDX, CHUNK), jnp.int32),
            pltpu.VMEM((NIDX, CHUNK), jnp.int32),
            [pltpu.SemaphoreType.DMA] * NBUF,
        ],
    )
    def conv_kernel(m_d_hbm, m_s_hbm, eid_hbm, eis_hbm, agg_d_hbm, agg_s_hbm,
                    msp, acc, rows, src_idx, dst_idx, gsems):
        c = lax.axis_index("c")
        s = lax.axis_index("s")

        def zrow(i, _):
            def zlane(j, _):
                rows[0][i, pl.ds(j * BLANES, BLANES)] = jnp.zeros((BLANES,), jnp.bfloat16)
                return 0
            return lax.fori_loop(0, H // BLANES, zlane, 0)
        lax.fori_loop(0, CHUNK, zrow, 0)

        def run(m_hbm, ei_hbm, out_hbm):
            sbase = s * STRIPE

            @pl.when(s < 15)
            def _():
                pltpu.sync_copy(m_hbm.at[pl.ds(sbase, STRIPE)],
                                msp.at[pl.ds(sbase, STRIPE)])

            @pl.when(s == 15)
            def _():
                pltpu.sync_copy(m_hbm.at[pl.ds(9600, 400)],
                                msp.at[pl.ds(9600, 400)])

            def zcopy(k, _):
                pltpu.sync_copy(rows[0], acc.at[pl.ds(sbase + k * CHUNK, CHUNK)])
                return 0
            lax.fori_loop(0, STRIPE // CHUNK, zcopy, 0)
            plsc.subcore_barrier()

            def gather(k, b):
                return pltpu.make_async_copy(msp.at[src_idx.at[k]],
                                             rows[b], gsems[b])

            def grp(gi, _):
                pltpu.sync_copy(ei_hbm.at[0, s, pl.ds(gi * NIDX, NIDX)], src_idx)
                pltpu.sync_copy(ei_hbm.at[1, s, pl.ds(gi * NIDX, NIDX)], dst_idx)
                for b in range(NBUF):
                    gather(b, b).start()

                def inner(t, _):
                    for b in range(NBUF):
                        k = t * NBUF + b
                        gather(k, b).wait()
                        pltpu.sync_copy(rows[b], acc.at[dst_idx.at[k]], add=True)

                        @pl.when(k + NBUF < NIDX)
                        def _():
                            gather(k + NBUF, b).start()
                    return 0
                lax.fori_loop(0, NIDX // NBUF, inner, 0)
                return 0
            lax.fori_loop(0, NGRP, grp, 0)
            plsc.subcore_barrier()

            @pl.when(s < 15)
            def _():
                pltpu.sync_copy(acc.at[pl.ds(sbase, STRIPE)],
                                out_hbm.at[pl.ds(sbase, STRIPE)])

            @pl.when(s == 15)
            def _():
                pltpu.sync_copy(acc.at[pl.ds(9600, 400)],
                                out_hbm.at[pl.ds(9600, 400)])

        @pl.when(c == 0)
        def _():
            run(m_d_hbm, eid_hbm, agg_d_hbm)

        @pl.when(c == 1)
        def _():
            run(m_s_hbm, eis_hbm, agg_s_hbm)

    return conv_kernel(m_d, m_s, eid, eis)


def _pad_edges(ei):
    pad = EPAD - E
    pad_cols = jnp.concatenate([
        jnp.zeros((1, pad), jnp.int32),
        jnp.full((1, pad), PADROW, jnp.int32),
    ], axis=0)
    padded = jnp.concatenate([ei, pad_cols], axis=1)
    return padded.reshape(2, NUM_TILES, CPT, CHUNK)


def kernel(drug_x, drug_edge_index, dis_x, dis_edge_index,
           W1d, b1d, W2d, b2d, W1s, b1s, W2s, b2s):
    eid = _pad_edges(drug_edge_index)
    eis = _pad_edges(dis_edge_index)

    rows_blk = 2000
    m1_d, m1_s = _matmul2_bf16(drug_x, dis_x, W1d, W1s, rows_blk)
    a1_d, a1_s = _sc_conv(m1_d, m1_s, eid, eis)
    m2_d, m2_s = _bias_relu_matmul2_bf16(a1_d, a1_s, b1d, b1s, W2d, W2s, rows_blk)
    a2_d, a2_s = _sc_conv(m2_d, m2_s, eid, eis)
    emb1, emb2 = _bias_add2_f32(a2_d, a2_s, b2d, b2s, rows_blk)
    return (emb1, emb2)
